# baseline (speedup 1.0000x reference)
import functools

import jax
import jax.numpy as jnp
import numpy as np
from jax import lax
from jax.experimental import pallas as pl
from jax.experimental.pallas import tpu as pltpu
from jax.experimental.pallas import tpu_sc as plsc

B = 32
V = 1_000_000
NCAND = 64
TEMP = np.float32(0.7)
INV_TEMP = np.float32(1.0 / 0.7)
TOP_P = np.float32(0.9)
TOP_K = 50
NEG_INF = np.float32(-np.inf)
EXP_SHIFT = np.float32(7.0)
TINY = np.float32(np.finfo(np.float32).tiny)
BIGI = np.int32(2**31 - 1)

CHUNK = 8000
NCHUNK = 125
GROUP = 320
JV = 20
GPC = 25
NGROUP = 3125
L2PAD = 3328
NL3 = 196
L3PAD = 208


def _group_max(buf, off):
    m = buf[pl.ds(off, 16)]
    for j in range(1, JV):
        m = jnp.maximum(m, buf[pl.ds(off + 16 * j, 16)])
    return m


_IOTA = None


def _set1(ref, idx, val, iota):
    off = pl.multiple_of((idx // 16) * 16, 16)
    lane = idx % 16
    v = ref[pl.ds(off, 16)]
    ref[pl.ds(off, 16)] = jnp.where(iota == lane, val, v)


def _get1_i32(ref, idx, iota):
    off = pl.multiple_of((idx // 16) * 16, 16)
    lane = idx % 16
    v = ref[pl.ds(off, 16)]
    return jnp.min(jnp.where(iota == lane, v, BIGI))


def _sc_body(logits_hbm, outv_hbm, outi_hbm,
             buf_a, buf_b, l1, l2, l3, gbuf, candv, candi,
             sem_a, sem_b):
    row = lax.axis_index("s") * 2 + lax.axis_index("c")
    iota = lax.iota(jnp.int32, 16)

    rbase = row * V

    def start(c, buf, sem):
        return pltpu.async_copy(
            logits_hbm.at[pl.ds(rbase + c * CHUNK, CHUNK)], buf, sem)

    def wait(buf, sem):
        pltpu.make_async_copy(
            logits_hbm.at[pl.ds(0, CHUNK)], buf, sem).wait()

    ninf = jnp.full((16,), NEG_INF, jnp.float32)

    def init_l2(k, _):
        l2[pl.ds(pl.multiple_of(k * 16, 16), 16)] = ninf
        return 0
    lax.fori_loop(0, L2PAD // 16, init_l2, 0)
    for k in range(L3PAD // 16):
        l3[pl.ds(k * 16, 16)] = ninf

    def process_chunk(c, buf):
        def group_body(g, _):
            off = pl.multiple_of(g * GROUP, 16)
            m = _group_max(buf, off)
            gg = c * GPC + g
            l1[pl.ds(pl.multiple_of(gg * 16, 16), 16)] = m
            _set1(l2, gg, jnp.max(m), iota)
            return 0
        lax.fori_loop(0, GPC, group_body, 0)

    start(0, buf_a, sem_a)

    def chunk_pair(i, _):
        wait(buf_a, sem_a)
        start(2 * i + 1, buf_b, sem_b)
        process_chunk(2 * i, buf_a)
        wait(buf_b, sem_b)
        start(2 * i + 2, buf_a, sem_a)
        process_chunk(2 * i + 1, buf_b)
        return 0
    lax.fori_loop(0, (NCHUNK - 1) // 2, chunk_pair, 0)
    wait(buf_a, sem_a)
    process_chunk(NCHUNK - 1, buf_a)

    def l3_body(q, _):
        v = l2[pl.ds(pl.multiple_of(q * 16, 16), 16)]
        _set1(l3, q, jnp.max(v), iota)
        return 0
    lax.fori_loop(0, NL3, l3_body, 0)

    def extract(t, _):
        mv = l3[pl.ds(0, 16)]
        for k in range(1, L3PAD // 16):
            mv = jnp.maximum(mv, l3[pl.ds(k * 16, 16)])
        m3 = jnp.max(mv)
        i3 = jnp.full((16,), BIGI, jnp.int32)
        for k in range(L3PAD // 16):
            vk = l3[pl.ds(k * 16, 16)]
            i3 = jnp.minimum(i3, jnp.where(vk == m3, iota + 16 * k, BIGI))
        t3 = jnp.min(i3)
        v2 = l2[pl.ds(pl.multiple_of(t3 * 16, 16), 16)]
        m2 = jnp.max(v2)
        g = t3 * 16 + jnp.min(jnp.where(v2 == m2, iota, BIGI))
        v1 = l1[pl.ds(pl.multiple_of(g * 16, 16), 16)]
        m1 = jnp.max(v1)
        lane = jnp.min(jnp.where(v1 == m1, iota, BIGI))
        pltpu.sync_copy(logits_hbm.at[pl.ds(rbase + g * GROUP, GROUP)], gbuf)

        def remask(p, _):
            cp = _get1_i32(candi, p, iota)
            gp = cp // GROUP

            @pl.when(gp == g)
            def _():
                _set1(gbuf, cp - g * GROUP, NEG_INF, iota)
            return 0
        lax.fori_loop(0, t, remask, 0)
        sel = jnp.full((16,), BIGI, jnp.int32)
        for j in range(JV):
            vj = gbuf[pl.ds(j * 16, 16)]
            sel = jnp.minimum(sel, jnp.where(vj == m1, np.int32(j), BIGI))
        jwin = jnp.min(jnp.where(iota == lane, sel, BIGI))
        col = g * GROUP + lane + 16 * jwin
        _set1(candv, t, m1, iota)
        _set1(candi, t, col, iota)
        _set1(gbuf, lane + 16 * jwin, NEG_INF, iota)
        m = _group_max(gbuf, 0)
        l1[pl.ds(pl.multiple_of(g * 16, 16), 16)] = m
        _set1(l2, g, jnp.max(m), iota)
        v2n = l2[pl.ds(pl.multiple_of(t3 * 16, 16), 16)]
        _set1(l3, t3, jnp.max(v2n), iota)
        return 0
    lax.fori_loop(0, NCAND, extract, 0)

    pltpu.sync_copy(candv, outv_hbm.at[row])
    pltpu.sync_copy(candi, outi_hbm.at[row])


@jax.jit
def _sc_topk(logits):
    mesh = plsc.VectorSubcoreMesh(
        core_axis_name="c", subcore_axis_name="s",
        num_cores=2, num_subcores=16)
    fn = pl.kernel(
        _sc_body,
        out_type=(jax.ShapeDtypeStruct((B, NCAND), jnp.float32),
                  jax.ShapeDtypeStruct((B, NCAND), jnp.int32)),
        mesh=mesh,
        compiler_params=pltpu.CompilerParams(
            use_tc_tiling_on_sc=False, needs_layout_passes=False),
        scratch_types=[
            pltpu.VMEM((CHUNK,), jnp.float32),
            pltpu.VMEM((CHUNK,), jnp.float32),
            pltpu.VMEM((NGROUP * 16,), jnp.float32),
            pltpu.VMEM((L2PAD,), jnp.float32),
            pltpu.VMEM((L3PAD,), jnp.float32),
            pltpu.VMEM((GROUP,), jnp.float32),
            pltpu.VMEM((NCAND,), jnp.float32),
            pltpu.VMEM((NCAND,), jnp.int32),
            pltpu.SemaphoreType.DMA,
            pltpu.SemaphoreType.DMA,
        ],
    )
    return fn(logits)


WBLK = 65536
NBLK = 16


def _rotl(x, r):
    return lax.shift_left(x, np.int32(r)) | lax.shift_right_logical(
        x, np.int32(32 - r))


def _threefry_bits(x1):
    ks0 = np.int32(0)
    ks1 = np.int32(42)
    ks2 = np.int32(0 ^ 42 ^ 0x1BD11BDA)
    ks = [ks0, ks1, ks2]
    rotations = [[13, 15, 26, 6], [17, 29, 16, 24]]
    x0 = jnp.zeros_like(x1) + ks0
    x1 = x1 + ks1
    for i in range(5):
        for r in rotations[i % 2]:
            x0 = x0 + x1
            x1 = _rotl(x1, r)
            x1 = x1 ^ x0
        x0 = x0 + ks[(i + 1) % 3]
        x1 = x1 + ks[(i + 2) % 3] + np.int32(i + 1)
    return x0 ^ x1


def _tc_body(logits_ref, candv_ref, candi_ref, tok_ref, acc_ref):
    pid = pl.program_id(0)

    @pl.when(pid == 0)
    def _():
        acc_ref[...] = jnp.zeros_like(acc_ref)

    x = logits_ref[...]
    col = pid * WBLK + lax.broadcasted_iota(jnp.int32, x.shape, 1)
    e = jnp.where(col < V, jnp.exp((x - EXP_SHIFT) * INV_TEMP), 0.0)
    acc_ref[...] += jnp.broadcast_to(
        jnp.sum(e, axis=1, keepdims=True), acc_ref.shape)

    @pl.when(pid == NBLK - 1)
    def _():
        s = acc_ref[:, 0:1]
        v = candv_ref[...]
        ci = candi_ref[...]
        d = v / TEMP
        ec = jnp.exp((v - EXP_SHIFT) * INV_TEMP)
        jj = lax.broadcasted_iota(jnp.int32, (NCAND, NCAND), 0)
        rr = lax.broadcasted_iota(jnp.int32, (NCAND, NCAND), 1)
        tri = (jj < rr).astype(jnp.float32)
        prefix = jnp.dot(ec, tri, preferred_element_type=jnp.float32)
        keep_p = prefix < TOP_P * s
        rcnt = jnp.sum(keep_p.astype(jnp.int32), axis=1, keepdims=True)
        kth = jnp.where(rcnt >= TOP_K, d[:, TOP_K - 1:TOP_K], NEG_INF)
        keep = keep_p & (d >= kth)
        flat = lax.broadcasted_iota(jnp.int32, ci.shape, 0) * V + ci
        bits = _threefry_bits(flat)
        fb = lax.shift_right_logical(bits, np.int32(9)) | np.int32(0x3F800000)
        floats = lax.bitcast_convert_type(fb, jnp.float32) - np.float32(1.0)
        u = jnp.maximum(TINY, floats + TINY)
        gum = -jnp.log(-jnp.log(u))
        score = jnp.where(keep, d + gum, NEG_INF)
        mx = jnp.max(score, axis=1, keepdims=True)
        tok = jnp.min(jnp.where(score == mx, ci, BIGI), axis=1)
        tok_ref[...] = jnp.broadcast_to(tok[:, None], tok_ref.shape)


@jax.jit
def _tc_sample(logits, candv, candi):
    return pl.pallas_call(
        _tc_body,
        grid=(NBLK,),
        in_specs=[
            pl.BlockSpec((B, WBLK), lambda i: (0, i)),
            pl.BlockSpec((B, NCAND), lambda i: (0, 0)),
            pl.BlockSpec((B, NCAND), lambda i: (0, 0)),
        ],
        out_specs=pl.BlockSpec((B, 128), lambda i: (0, 0)),
        out_shape=jax.ShapeDtypeStruct((B, 128), jnp.int32),
        scratch_shapes=[pltpu.VMEM((B, 128), jnp.float32)],
    )(logits, candv, candi)


def kernel(input_ids, logits, input_metadata):
    candv, candi = _sc_topk(jnp.reshape(logits, (B * V,)))
    tok = _tc_sample(logits, candv, candi)
    return tok[:, 0]


# speedup vs baseline: 1.0004x; 1.0004x over previous
import jax
import jax.numpy as jnp
import numpy as np
from jax import lax
from jax.experimental import pallas as pl
from jax.experimental.pallas import tpu as pltpu
from jax.experimental.pallas import tpu_sc as plsc

B = 32
V = 1_000_000
NCAND = 64
TEMP = np.float32(0.7)
INV_TEMP = np.float32(1.0 / 0.7)
TOP_P = np.float32(0.9)
TOP_K = 50
NEG_INF = np.float32(-np.inf)
EXP_SHIFT = np.float32(7.0)
TINY = np.float32(np.finfo(np.float32).tiny)
BIGI = np.int32(2**31 - 1)

CHUNK = 8000
NCHUNK = 125
GROUP = 320
JV = 20
GPC = 25
NGROUP = 3125
L2PAD = 3328
NL3 = 196
L3PAD = 208


def _group_max(buf, off):
    m = buf[pl.ds(off, 16)]
    for j in range(1, JV):
        m = jnp.maximum(m, buf[pl.ds(off + 16 * j, 16)])
    return m


def _set1(ref, idx, val, iota):
    off = pl.multiple_of((idx // 16) * 16, 16)
    lane = idx % 16
    v = ref[pl.ds(off, 16)]
    ref[pl.ds(off, 16)] = jnp.where(iota == lane, val, v)


def _get1_i32(ref, idx, iota):
    off = pl.multiple_of((idx // 16) * 16, 16)
    lane = idx % 16
    v = ref[pl.ds(off, 16)]
    return jnp.min(jnp.where(iota == lane, v, BIGI))


def _sc_body(logits_hbm, outv_hbm, outi_hbm,
             buf_a, buf_b, l1, l2, l3, gbuf, candv, candi,
             sem_a, sem_b):
    row = lax.axis_index("s") * 2 + lax.axis_index("c")
    iota = lax.iota(jnp.int32, 16)

    rbase = row * V

    def start(c, buf, sem):
        return pltpu.async_copy(
            logits_hbm.at[pl.ds(rbase + c * CHUNK, CHUNK)], buf, sem)

    def wait(buf, sem):
        pltpu.make_async_copy(
            logits_hbm.at[pl.ds(0, CHUNK)], buf, sem).wait()

    ninf = jnp.full((16,), NEG_INF, jnp.float32)

    def init_l2(k, _):
        l2[pl.ds(pl.multiple_of(k * 16, 16), 16)] = ninf
        return 0
    lax.fori_loop(0, L2PAD // 16, init_l2, 0)
    for k in range(L3PAD // 16):
        l3[pl.ds(k * 16, 16)] = ninf

    def process_chunk(c, buf):
        def group_body(g, _):
            off = pl.multiple_of(g * GROUP, 16)
            m = _group_max(buf, off)
            gg = c * GPC + g
            l1[pl.ds(pl.multiple_of(gg * 16, 16), 16)] = m
            _set1(l2, gg, jnp.max(m), iota)
            return 0
        lax.fori_loop(0, GPC, group_body, 0)

    start(0, buf_a, sem_a)

    def chunk_pair(i, _):
        wait(buf_a, sem_a)
        start(2 * i + 1, buf_b, sem_b)
        process_chunk(2 * i, buf_a)
        wait(buf_b, sem_b)
        start(2 * i + 2, buf_a, sem_a)
        process_chunk(2 * i + 1, buf_b)
        return 0
    lax.fori_loop(0, (NCHUNK - 1) // 2, chunk_pair, 0)
    wait(buf_a, sem_a)
    process_chunk(NCHUNK - 1, buf_a)

    def l3_body(q, _):
        v = l2[pl.ds(pl.multiple_of(q * 16, 16), 16)]
        _set1(l3, q, jnp.max(v), iota)
        return 0
    lax.fori_loop(0, NL3, l3_body, 0)

    def extract(t, _):
        mv = l3[pl.ds(0, 16)]
        for k in range(1, L3PAD // 16):
            mv = jnp.maximum(mv, l3[pl.ds(k * 16, 16)])
        m3 = jnp.max(mv)
        i3 = jnp.full((16,), BIGI, jnp.int32)
        for k in range(L3PAD // 16):
            vk = l3[pl.ds(k * 16, 16)]
            i3 = jnp.minimum(i3, jnp.where(vk == m3, iota + 16 * k, BIGI))
        t3 = jnp.min(i3)
        v2 = l2[pl.ds(pl.multiple_of(t3 * 16, 16), 16)]
        m2 = jnp.max(v2)
        g = t3 * 16 + jnp.min(jnp.where(v2 == m2, iota, BIGI))
        v1 = l1[pl.ds(pl.multiple_of(g * 16, 16), 16)]
        m1 = jnp.max(v1)
        lane = jnp.min(jnp.where(v1 == m1, iota, BIGI))
        pltpu.sync_copy(logits_hbm.at[pl.ds(rbase + g * GROUP, GROUP)], gbuf)

        def remask(p, _):
            cp = _get1_i32(candi, p, iota)
            gp = cp // GROUP

            @pl.when(gp == g)
            def _():
                _set1(gbuf, cp - g * GROUP, NEG_INF, iota)
            return 0
        lax.fori_loop(0, t, remask, 0)
        sel = jnp.full((16,), BIGI, jnp.int32)
        for j in range(JV):
            vj = gbuf[pl.ds(j * 16, 16)]
            sel = jnp.minimum(sel, jnp.where(vj == m1, np.int32(j), BIGI))
        jwin = jnp.min(jnp.where(iota == lane, sel, BIGI))
        col = g * GROUP + lane + 16 * jwin
        _set1(candv, t, m1, iota)
        _set1(candi, t, col, iota)
        _set1(gbuf, lane + 16 * jwin, NEG_INF, iota)
        m = _group_max(gbuf, 0)
        l1[pl.ds(pl.multiple_of(g * 16, 16), 16)] = m
        _set1(l2, g, jnp.max(m), iota)
        v2n = l2[pl.ds(pl.multiple_of(t3 * 16, 16), 16)]
        _set1(l3, t3, jnp.max(v2n), iota)
        return 0
    lax.fori_loop(0, NCAND, extract, 0)

    pltpu.sync_copy(candv, outv_hbm.at[row])
    pltpu.sync_copy(candi, outi_hbm.at[row])


@jax.jit
def _sc_topk(logits):
    mesh = plsc.VectorSubcoreMesh(
        core_axis_name="c", subcore_axis_name="s",
        num_cores=2, num_subcores=16)
    fn = pl.kernel(
        _sc_body,
        out_type=(jax.ShapeDtypeStruct((B, NCAND), jnp.float32),
                  jax.ShapeDtypeStruct((B, NCAND), jnp.int32)),
        mesh=mesh,
        compiler_params=pltpu.CompilerParams(
            use_tc_tiling_on_sc=False, needs_layout_passes=False),
        scratch_types=[
            pltpu.VMEM((CHUNK,), jnp.float32),
            pltpu.VMEM((CHUNK,), jnp.float32),
            pltpu.VMEM((NGROUP * 16,), jnp.float32),
            pltpu.VMEM((L2PAD,), jnp.float32),
            pltpu.VMEM((L3PAD,), jnp.float32),
            pltpu.VMEM((GROUP,), jnp.float32),
            pltpu.VMEM((NCAND,), jnp.float32),
            pltpu.VMEM((NCAND,), jnp.int32),
            pltpu.SemaphoreType.DMA,
            pltpu.SemaphoreType.DMA,
        ],
    )
    return fn(logits)


WBLK = 65536
NBLK = 16


def _rotl(x, r):
    return lax.shift_left(x, np.int32(r)) | lax.shift_right_logical(
        x, np.int32(32 - r))


def _threefry_bits(x1):
    ks0 = np.int32(0)
    ks1 = np.int32(42)
    ks2 = np.int32(0 ^ 42 ^ 0x1BD11BDA)
    ks = [ks0, ks1, ks2]
    rotations = [[13, 15, 26, 6], [17, 29, 16, 24]]
    x0 = jnp.zeros_like(x1) + ks0
    x1 = x1 + ks1
    for i in range(5):
        for r in rotations[i % 2]:
            x0 = x0 + x1
            x1 = _rotl(x1, r)
            x1 = x1 ^ x0
        x0 = x0 + ks[(i + 1) % 3]
        x1 = x1 + ks[(i + 2) % 3] + np.int32(i + 1)
    return x0 ^ x1


def _tc_body(logits_ref, candv_ref, candi_ref, tok_ref, acc_ref):
    pid = pl.program_id(0)

    @pl.when(pid == 0)
    def _():
        acc_ref[...] = jnp.zeros_like(acc_ref)

    x = logits_ref[...]
    col = pid * WBLK + lax.broadcasted_iota(jnp.int32, x.shape, 1)
    e = jnp.where(col < V, jnp.exp((x - EXP_SHIFT) * INV_TEMP), 0.0)
    acc_ref[...] += jnp.broadcast_to(
        jnp.sum(e, axis=1, keepdims=True), acc_ref.shape)

    @pl.when(pid == NBLK - 1)
    def _():
        s = acc_ref[:, 0:1]
        v = candv_ref[...]
        ci = candi_ref[...]
        d = v / TEMP
        ec = jnp.exp((v - EXP_SHIFT) * INV_TEMP)
        jj = lax.broadcasted_iota(jnp.int32, (NCAND, NCAND), 0)
        rr = lax.broadcasted_iota(jnp.int32, (NCAND, NCAND), 1)
        tri = (jj < rr).astype(jnp.float32)
        prefix = jnp.dot(ec, tri, preferred_element_type=jnp.float32)
        keep_p = prefix < TOP_P * s
        rcnt = jnp.sum(keep_p.astype(jnp.int32), axis=1, keepdims=True)
        kth = jnp.where(rcnt >= TOP_K, d[:, TOP_K - 1:TOP_K], NEG_INF)
        keep = keep_p & (d >= kth)
        flat = lax.broadcasted_iota(jnp.int32, ci.shape, 0) * V + ci
        bits = _threefry_bits(flat)
        fb = lax.shift_right_logical(bits, np.int32(9)) | np.int32(0x3F800000)
        floats = lax.bitcast_convert_type(fb, jnp.float32) - np.float32(1.0)
        u = jnp.maximum(TINY, floats + TINY)
        gum = -jnp.log(-jnp.log(u))
        score = jnp.where(keep, d + gum, NEG_INF)
        mx = jnp.max(score, axis=1, keepdims=True)
        tok = jnp.min(jnp.where(score == mx, ci, BIGI), axis=1)
        tok_ref[...] = jnp.broadcast_to(tok[:, None], tok_ref.shape)


@jax.jit
def _tc_sample(logits, candv, candi):
    return pl.pallas_call(
        _tc_body,
        grid=(NBLK,),
        in_specs=[
            pl.BlockSpec((B, WBLK), lambda i: (0, i)),
            pl.BlockSpec((B, NCAND), lambda i: (0, 0)),
            pl.BlockSpec((B, NCAND), lambda i: (0, 0)),
        ],
        out_specs=pl.BlockSpec((B, 128), lambda i: (0, 0)),
        out_shape=jax.ShapeDtypeStruct((B, 128), jnp.int32),
        scratch_shapes=[pltpu.VMEM((B, 128), jnp.float32)],
    )(logits, candv, candi)


def kernel(input_ids, logits, input_metadata):
    candv, candi = _sc_topk(jnp.reshape(logits, (B * V,)))
    tok = _tc_sample(logits, candv, candi)
    return tok[:, 0]
